# baseline (speedup 1.0000x reference)
import functools

import jax
import jax.numpy as jnp
from jax import lax
from jax.experimental import pallas as pl
from jax.experimental.pallas import tpu as pltpu
from jax.experimental.pallas import tpu_sc as plsc

_MAX_POS = 8192
_EMBEDDING_DIM = 16
_NUM_CORES = 2
_NUM_SUBCORES = 16
_NUM_WORKERS = _NUM_CORES * _NUM_SUBCORES
_ROWS_PER_WORKER = _MAX_POS // _NUM_WORKERS


@functools.partial(
    pl.kernel,
    out_type=jax.ShapeDtypeStruct((_MAX_POS, _EMBEDDING_DIM), jnp.float32),
    mesh=plsc.VectorSubcoreMesh(core_axis_name="c", subcore_axis_name="s"),
)
def _sc_row_copy(w_hbm, out_hbm):
    wid = lax.axis_index("s") * _NUM_CORES + lax.axis_index("c")
    base = wid * _ROWS_PER_WORKER
    pltpu.sync_copy(w_hbm.at[pl.ds(base, _ROWS_PER_WORKER)],
                    out_hbm.at[pl.ds(base, _ROWS_PER_WORKER)])


def kernel(input, weights):
    del input
    return _sc_row_copy(weights)


# speedup vs baseline: 5.3061x; 5.3061x over previous
import functools

import jax
import jax.numpy as jnp
from jax import lax
from jax.experimental import pallas as pl
from jax.experimental.pallas import tpu as pltpu
from jax.experimental.pallas import tpu_sc as plsc

_MAX_POS = 8192
_EMBEDDING_DIM = 16
_NUM_CORES = 2
_NUM_SUBCORES = 16
_NUM_WORKERS = _NUM_CORES * _NUM_SUBCORES
_ROWS_PER_WORKER = _MAX_POS // _NUM_WORKERS


_ROWS_PER_CORE = _MAX_POS // _NUM_CORES


@functools.partial(
    pl.kernel,
    out_type=jax.ShapeDtypeStruct((_MAX_POS, _EMBEDDING_DIM), jnp.float32),
    mesh=plsc.ScalarSubcoreMesh(axis_name="c", num_cores=_NUM_CORES),
    scratch_types=[
        pltpu.MemorySpace.VMEM_SHARED((_ROWS_PER_CORE, _EMBEDDING_DIM), jnp.float32)
    ],
)
def _sc_row_copy(w_hbm, out_hbm, spmem):
    cid = lax.axis_index("c")
    base = cid * _ROWS_PER_CORE
    pltpu.sync_copy(w_hbm.at[pl.ds(base, _ROWS_PER_CORE)], spmem)
    pltpu.sync_copy(spmem, out_hbm.at[pl.ds(base, _ROWS_PER_CORE)])


def kernel(input, weights):
    del input
    return _sc_row_copy(weights)


# speedup vs baseline: 5.5071x; 1.0379x over previous
import functools

import jax
import jax.numpy as jnp
from jax import lax
from jax.experimental import pallas as pl
from jax.experimental.pallas import tpu as pltpu
from jax.experimental.pallas import tpu_sc as plsc

_MAX_POS = 8192
_EMBEDDING_DIM = 16
_NUM_CORES = 2
_NUM_SUBCORES = 16
_NUM_WORKERS = _NUM_CORES * _NUM_SUBCORES
_ROWS_PER_WORKER = _MAX_POS // _NUM_WORKERS


_ROWS_PER_TILE = _MAX_POS // _NUM_SUBCORES


@functools.partial(
    pl.kernel,
    out_type=jax.ShapeDtypeStruct((_MAX_POS, _EMBEDDING_DIM), jnp.float32),
    mesh=plsc.VectorSubcoreMesh(
        core_axis_name="c", subcore_axis_name="s", num_cores=1
    ),
    scratch_types=[pltpu.VMEM((_ROWS_PER_TILE, _EMBEDDING_DIM), jnp.float32)],
)
def _sc_row_copy(w_hbm, out_hbm, rows_v):
    sid = lax.axis_index("s")
    base = sid * _ROWS_PER_TILE
    pltpu.sync_copy(w_hbm.at[pl.ds(base, _ROWS_PER_TILE)], rows_v)
    pltpu.sync_copy(rows_v, out_hbm.at[pl.ds(base, _ROWS_PER_TILE)])


def kernel(input, weights):
    del input
    return _sc_row_copy(weights)


# speedup vs baseline: 5.6169x; 1.0199x over previous
import functools

import jax
import jax.numpy as jnp
from jax import lax
from jax.experimental import pallas as pl
from jax.experimental.pallas import tpu as pltpu
from jax.experimental.pallas import tpu_sc as plsc

_MAX_POS = 8192
_EMBEDDING_DIM = 16
_NUM_CORES = 2
_NUM_SUBCORES = 16
_NUM_WORKERS = _NUM_CORES * _NUM_SUBCORES
_ROWS_PER_WORKER = _MAX_POS // _NUM_WORKERS


_HALF_ROWS = _ROWS_PER_WORKER // 2


@functools.partial(
    pl.kernel,
    out_type=jax.ShapeDtypeStruct((_MAX_POS, _EMBEDDING_DIM), jnp.float32),
    mesh=plsc.VectorSubcoreMesh(core_axis_name="c", subcore_axis_name="s"),
    scratch_types=[
        pltpu.VMEM((_HALF_ROWS, _EMBEDDING_DIM), jnp.float32),
        pltpu.VMEM((_HALF_ROWS, _EMBEDDING_DIM), jnp.float32),
        pltpu.SemaphoreType.DMA,
        pltpu.SemaphoreType.DMA,
    ],
)
def _sc_row_copy(w_hbm, out_hbm, buf0, buf1, sem0, sem1):
    wid = lax.axis_index("s") * _NUM_CORES + lax.axis_index("c")
    base = wid * _ROWS_PER_WORKER
    in0 = pltpu.async_copy(w_hbm.at[pl.ds(base, _HALF_ROWS)], buf0, sem0)
    in1 = pltpu.async_copy(
        w_hbm.at[pl.ds(base + _HALF_ROWS, _HALF_ROWS)], buf1, sem1
    )
    in0.wait()
    out0 = pltpu.async_copy(buf0, out_hbm.at[pl.ds(base, _HALF_ROWS)], sem0)
    in1.wait()
    out1 = pltpu.async_copy(
        buf1, out_hbm.at[pl.ds(base + _HALF_ROWS, _HALF_ROWS)], sem1
    )
    out0.wait()
    out1.wait()


def kernel(input, weights):
    del input
    return _sc_row_copy(weights)


# speedup vs baseline: 5.7584x; 1.0252x over previous
import functools

import jax
import jax.numpy as jnp
from jax import lax
from jax.experimental import pallas as pl
from jax.experimental.pallas import tpu as pltpu
from jax.experimental.pallas import tpu_sc as plsc

_MAX_POS = 8192
_EMBEDDING_DIM = 16
_NUM_CORES = 2
_NUM_SUBCORES = 16
_NUM_WORKERS = _NUM_CORES * _NUM_SUBCORES
_ROWS_PER_WORKER = _MAX_POS // _NUM_WORKERS


_TOTAL = _MAX_POS * _EMBEDDING_DIM
_PER_WORKER = _TOTAL // _NUM_WORKERS


@functools.partial(
    pl.kernel,
    out_type=jax.ShapeDtypeStruct((_TOTAL,), jnp.float32),
    mesh=plsc.VectorSubcoreMesh(core_axis_name="c", subcore_axis_name="s"),
    scratch_types=[pltpu.VMEM((_PER_WORKER,), jnp.float32)],
)
def _sc_row_copy(w_hbm, out_hbm, buf):
    wid = lax.axis_index("s") * _NUM_CORES + lax.axis_index("c")
    base = wid * _PER_WORKER
    pltpu.sync_copy(w_hbm.at[pl.ds(base, _PER_WORKER)], buf)
    pltpu.sync_copy(buf, out_hbm.at[pl.ds(base, _PER_WORKER)])


def kernel(input, weights):
    del input
    flat = _sc_row_copy(weights.reshape(_TOTAL))
    return flat.reshape(_MAX_POS, _EMBEDDING_DIM)
